# baseline (speedup 1.0000x reference)
import functools

import numpy as np
import jax
import jax.numpy as jnp
from jax import lax
from jax.experimental import pallas as pl
from jax.experimental.pallas import tpu as pltpu
from jax.experimental.pallas import tpu_sc as plsc

N = 10000
E = 320000
D = 128
DE = 16
EPS = 0.0

NC = 2
NS = 16
L = 16
NW = NC * NS
EPT = E // NW
C = 80
NCHUNK = EPT // C
NP = 10240
RPT = NP // NS

_Q = np.zeros(D, dtype=np.int32)
for _t in range(D // 32):
    for _k in range(16):
        _Q[32 * _t + _k] = 16 * _t + _k
        _Q[32 * _t + 16 + _k] = D // 2 + 16 * _t + _k

_sc_mesh = plsc.VectorSubcoreMesh(
    core_axis_name="c", subcore_axis_name="s", num_cores=NC, num_subcores=NS
)


_BE2 = 1000


def _mlp_one(x, w1, b1, w2, b2, pa, pb):
    t = jnp.maximum(
        jnp.dot(x.astype(jnp.bfloat16), w1, preferred_element_type=jnp.float32) + b1,
        0.0,
    )
    y = (
        jnp.dot(t.astype(jnp.bfloat16), w2, preferred_element_type=jnp.float32) + b2
    ).astype(jnp.bfloat16)
    ya = jnp.dot(y, pa, preferred_element_type=jnp.float32).astype(jnp.bfloat16)
    yb = jnp.dot(y, pb, preferred_element_type=jnp.float32).astype(jnp.bfloat16)
    ua = jax.lax.bitcast_convert_type(ya, jnp.uint16).astype(jnp.int32)
    ub = jax.lax.bitcast_convert_type(yb, jnp.uint16).astype(jnp.int32)
    return ua | (ub << 16)


def _edge_mlp_body(xe_ref, xo_ref, w1_ref, b1_ref, w2_ref, b2_ref, pa_ref, pb_ref, o_ref):
    w1 = w1_ref[...]
    b1 = b1_ref[...]
    w2 = w2_ref[...]
    b2 = b2_ref[...]
    pa = pa_ref[...]
    pb = pb_ref[...]
    pe = _mlp_one(xe_ref[...], w1, b1, w2, b2, pa, pb)
    po = _mlp_one(xo_ref[...], w1, b1, w2, b2, pa, pb)
    o_ref[...] = jax.lax.bitcast_convert_type(
        jnp.concatenate([pe, po], axis=1), jnp.float32
    )


def _edge_mlp(ef3, w1, b1, w2, b2, pa, pb):
    return pl.pallas_call(
        _edge_mlp_body,
        grid=(E // 2 // _BE2,),
        in_specs=[
            pl.BlockSpec((_BE2, DE), lambda i: (i, 0)),
            pl.BlockSpec((_BE2, DE), lambda i: (i + E // 2 // _BE2, 0)),
            pl.BlockSpec((DE, D), lambda i: (0, 0)),
            pl.BlockSpec((1, D), lambda i: (0, 0)),
            pl.BlockSpec((D, D), lambda i: (0, 0)),
            pl.BlockSpec((1, D), lambda i: (0, 0)),
            pl.BlockSpec((D, D // 2), lambda i: (0, 0)),
            pl.BlockSpec((D, D // 2), lambda i: (0, 0)),
        ],
        out_specs=pl.BlockSpec((_BE2, D), lambda i: (i, 0)),
        out_shape=jax.ShapeDtypeStruct((E // 2, D), jnp.float32),
    )(ef3, ef3, w1, b1, w2, b2, pa, pb)


@functools.partial(
    pl.kernel,
    out_type=jax.ShapeDtypeStruct((NC, NP, D), jnp.float32),
    mesh=_sc_mesh,
    scratch_types=[
        pltpu.VMEM_SHARED((NP, D), jnp.float32),
        pltpu.VMEM((C, D), jnp.float32),
        pltpu.VMEM((C, D), jnp.float32),
        pltpu.VMEM((C, D), jnp.float32),
        pltpu.VMEM((C, D), jnp.float32),
        pltpu.VMEM((C,), jnp.int32),
        pltpu.VMEM((C,), jnp.int32),
        pltpu.VMEM((C,), jnp.int32),
        pltpu.VMEM((C,), jnp.int32),
        pltpu.SemaphoreType.DMA,
        pltpu.SemaphoreType.DMA,
        pltpu.SemaphoreType.DMA,
        pltpu.SemaphoreType.DMA,
        pltpu.SemaphoreType.DMA,
        pltpu.SemaphoreType.DMA,
        pltpu.SemaphoreType.DMA,
        pltpu.SemaphoreType.DMA,
        pltpu.SemaphoreType.DMA,
        pltpu.SemaphoreType.DMA,
    ],
)
def _sc_pass(
    e_hbm, h_hbm, src_hbm, dst_hbm, out_hbm,
    aggr, ebuf0, ebuf1, hbuf0, hbuf1,
    sibuf0, sibuf1, dbuf0, dbuf1,
    seme0, seme1, semg0, semg1, sems0, sems1, semi0, semi1, semd0, semd1,
):
    c = lax.axis_index("c")
    s = lax.axis_index("s")
    wid = s * NC + c

    ebufs = (ebuf0, ebuf1)
    hbufs = (hbuf0, hbuf1)
    sibufs = (sibuf0, sibuf1)
    dbufs = (dbuf0, dbuf1)
    semes = (seme0, seme1)
    semgs = (semg0, semg1)
    semss = (sems0, sems1)
    semis = (semi0, semi1)
    semds = (semd0, semd1)

    def _zrow(i, carry):
        for j in range(D // L):
            hbuf0[i, pl.ds(j * L, L)] = jnp.zeros((L,), jnp.float32)
        return carry

    lax.fori_loop(0, C, _zrow, 0)
    zbase = s * RPT
    for k in range(RPT // C):
        pltpu.sync_copy(hbuf0, aggr.at[pl.ds(zbase + k * C, C)])
    plsc.subcore_barrier()

    pbase = wid * (EPT // 2)

    def _lo(g):
        return pl.ds(pbase + g * (C // 2), C // 2)

    def _hi(g):
        return pl.ds(pbase + g * (C // 2) + E // 2, C // 2)

    def _idx_dma(g, p):
        pltpu.async_copy(src_hbm.at[_lo(g)], sibufs[p].at[pl.ds(0, C // 2)], semis[p])
        pltpu.async_copy(
            src_hbm.at[_hi(g)], sibufs[p].at[pl.ds(C // 2, C // 2)], semis[p]
        )

    def _issue(g, p, wait_scatter):
        pltpu.make_async_copy(
            src_hbm.at[_lo(g)], sibufs[p].at[pl.ds(0, C // 2)], semis[p]
        ).wait()
        pltpu.make_async_copy(
            src_hbm.at[_hi(g)], sibufs[p].at[pl.ds(C // 2, C // 2)], semis[p]
        ).wait()
        pltpu.async_copy(h_hbm.at[sibufs[p]], hbufs[p], semgs[p])
        if wait_scatter:
            pltpu.make_async_copy(ebufs[p], aggr.at[dbufs[p]], semss[p]).wait()
        pltpu.async_copy(dst_hbm.at[_lo(g)], dbufs[p].at[pl.ds(0, C // 2)], semds[p])
        pltpu.async_copy(
            dst_hbm.at[_hi(g)], dbufs[p].at[pl.ds(C // 2, C // 2)], semds[p]
        )
        pltpu.async_copy(
            e_hbm.at[pl.ds(pbase + g * (C // 2), C // 2)],
            ebufs[p].at[pl.ds(C // 2, C // 2)],
            semes[p],
        )

    def _compute(g, p, prefetch_g):
        pltpu.make_async_copy(
            e_hbm.at[pl.ds(pbase + g * (C // 2), C // 2)],
            ebufs[p].at[pl.ds(C // 2, C // 2)],
            semes[p],
        ).wait()
        pltpu.make_async_copy(h_hbm.at[sibufs[p]], hbufs[p], semgs[p]).wait()
        if prefetch_g is not None:
            _idx_dma(prefetch_g, p)

        def _rowpair(ii, inner):
            ve = [
                lax.bitcast_convert_type(
                    ebufs[p][C // 2 + ii, pl.ds(L * t, L)], jnp.int32
                )
                for t in range(2 * (D // 32))
            ]
            res = []
            for r in range(2):
                i = ii + r * (C // 2)
                for t in range(D // 32):
                    v = ve[4 * r + t]
                    lo = hbufs[p][i, pl.ds(32 * t, L)] + lax.bitcast_convert_type(
                        v << 16, jnp.float32
                    )
                    hi = hbufs[p][i, pl.ds(32 * t + L, L)] + lax.bitcast_convert_type(
                        v & jnp.int32(-65536), jnp.float32
                    )
                    res.append((i, t, jnp.maximum(lo, 0.0), jnp.maximum(hi, 0.0)))
            for i, t, vlo, vhi in res:
                ebufs[p][i, pl.ds(32 * t, L)] = vlo
                ebufs[p][i, pl.ds(32 * t + L, L)] = vhi
            return inner

        lax.fori_loop(0, C // 2, _rowpair, 0)
        pltpu.make_async_copy(
            dst_hbm.at[_lo(g)], dbufs[p].at[pl.ds(0, C // 2)], semds[p]
        ).wait()
        pltpu.make_async_copy(
            dst_hbm.at[_hi(g)], dbufs[p].at[pl.ds(C // 2, C // 2)], semds[p]
        ).wait()
        pltpu.async_copy(ebufs[p], aggr.at[dbufs[p]], semss[p], add=True)

    _idx_dma(0, 0)
    _idx_dma(1, 1)
    _issue(0, 0, False)
    _issue(1, 1, False)
    _compute(0, 0, 2)
    _issue(2, 0, True)
    _compute(1, 1, 3)

    def _pair(k, carry):
        g = 2 * k
        _issue(g + 1, 1, True)
        _compute(g, 0, g + 2)
        _issue(g + 2, 0, True)
        _compute(g + 1, 1, g + 3)
        return carry

    lax.fori_loop(1, (NCHUNK - 1) // 2 - 1, _pair, 0)
    _issue(NCHUNK - 2, 1, True)
    _compute(NCHUNK - 3, 0, NCHUNK - 1)
    _issue(NCHUNK - 1, 0, True)
    _compute(NCHUNK - 2, 1, None)
    _compute(NCHUNK - 1, 0, None)
    pltpu.make_async_copy(ebuf0, aggr.at[dbuf0], sems0).wait()
    pltpu.make_async_copy(ebuf1, aggr.at[dbuf1], sems1).wait()
    plsc.subcore_barrier()

    pltpu.sync_copy(
        aggr.at[pl.ds(zbase, RPT)], out_hbm.at[c, pl.ds(zbase, RPT), :]
    )


_BN = 1000


def _apply_body(h_ref, p_ref, w_ref, b_ref, o_ref):
    x = (1.0 + EPS) * h_ref[...] + p_ref[0] + p_ref[1]
    o_ref[...] = jnp.maximum(
        jnp.dot(x, w_ref[...], preferred_element_type=jnp.float32) + b_ref[...], 0.0
    )


def _apply(h, partials, w, b):
    return pl.pallas_call(
        _apply_body,
        grid=(N // _BN,),
        in_specs=[
            pl.BlockSpec((_BN, D), lambda i: (i, 0)),
            pl.BlockSpec((NC, _BN, D), lambda i: (0, i, 0)),
            pl.BlockSpec((D, D), lambda i: (0, 0)),
            pl.BlockSpec((1, D), lambda i: (0, 0)),
        ],
        out_specs=pl.BlockSpec((_BN, D), lambda i: (i, 0)),
        out_shape=jax.ShapeDtypeStruct((N, D), jnp.float32),
    )(h, partials, w, b)


def kernel(node_feats, edge_feats, edge_index, We1, be1, We2, be2, W0, b0, W1, b1):
    src = edge_index[0]
    dst = edge_index[1]
    eye = np.eye(D, dtype=np.float32)
    e32 = _edge_mlp(
        edge_feats,
        We1.astype(jnp.bfloat16),
        be1.reshape(1, D),
        We2.astype(jnp.bfloat16),
        be2.reshape(1, D),
        jnp.asarray(eye[:, : D // 2], dtype=jnp.bfloat16),
        jnp.asarray(eye[:, D // 2 :], dtype=jnp.bfloat16),
    )
    q = jnp.asarray(_Q)
    h = node_feats[:, q]
    layer_params = (
        (W0[q][:, q], b0[q]),
        (W1[q], b1),
    )
    for w, b in layer_params:
        partials = _sc_pass(e32, h, src, dst)
        h = _apply(h, partials, w, b.reshape(1, D))
    return h


# speedup vs baseline: 1.4597x; 1.4597x over previous
import functools

import numpy as np
import jax
import jax.numpy as jnp
from jax import lax
from jax.experimental import pallas as pl
from jax.experimental.pallas import tpu as pltpu
from jax.experimental.pallas import tpu_sc as plsc

N = 10000
E = 320000
D = 128
DE = 16
EPS = 0.0

NC = 2
NS = 16
L = 16
NW = NC * NS
EPT = E // NW
C = 80
NCHUNK = EPT // C
NP = 10240
RPT = NP // NS

_Q = np.zeros(D, dtype=np.int32)
for _t in range(D // 32):
    for _k in range(16):
        _Q[32 * _t + _k] = 16 * _t + _k
        _Q[32 * _t + 16 + _k] = D // 2 + 16 * _t + _k

_sc_mesh = plsc.VectorSubcoreMesh(
    core_axis_name="c", subcore_axis_name="s", num_cores=NC, num_subcores=NS
)


_BE2 = 6400


def _mlp_one(xt, w1, b1, w2, b2, pa, pb):
    t = jnp.maximum(
        lax.dot_general(
            xt.astype(jnp.bfloat16),
            w1,
            (((0,), (0,)), ((), ())),
            preferred_element_type=jnp.float32,
        )
        + b1,
        0.0,
    )
    y = (
        jnp.dot(t.astype(jnp.bfloat16), w2, preferred_element_type=jnp.float32) + b2
    ).astype(jnp.bfloat16)
    ya = jnp.dot(y, pa, preferred_element_type=jnp.float32).astype(jnp.bfloat16)
    yb = jnp.dot(y, pb, preferred_element_type=jnp.float32).astype(jnp.bfloat16)
    ua = jax.lax.bitcast_convert_type(ya, jnp.uint16).astype(jnp.int32)
    ub = jax.lax.bitcast_convert_type(yb, jnp.uint16).astype(jnp.int32)
    return ua | (ub << 16)


def _edge_mlp_body(xe_ref, xo_ref, w1_ref, b1_ref, w2_ref, b2_ref, pa_ref, pb_ref, o_ref):
    w1 = w1_ref[...]
    b1 = b1_ref[...]
    w2 = w2_ref[...]
    b2 = b2_ref[...]
    pa = pa_ref[...]
    pb = pb_ref[...]
    pe = _mlp_one(xe_ref[...], w1, b1, w2, b2, pa, pb)
    po = _mlp_one(xo_ref[...], w1, b1, w2, b2, pa, pb)
    o_ref[...] = jax.lax.bitcast_convert_type(
        jnp.concatenate([pe, po], axis=1), jnp.float32
    )


def _edge_mlp(ef3, w1, b1, w2, b2, pa, pb):
    return pl.pallas_call(
        _edge_mlp_body,
        grid=(E // 2 // _BE2,),
        in_specs=[
            pl.BlockSpec((DE, _BE2), lambda i: (0, i)),
            pl.BlockSpec((DE, _BE2), lambda i: (0, i + E // 2 // _BE2)),
            pl.BlockSpec((DE, D), lambda i: (0, 0)),
            pl.BlockSpec((1, D), lambda i: (0, 0)),
            pl.BlockSpec((D, D), lambda i: (0, 0)),
            pl.BlockSpec((1, D), lambda i: (0, 0)),
            pl.BlockSpec((D, D // 2), lambda i: (0, 0)),
            pl.BlockSpec((D, D // 2), lambda i: (0, 0)),
        ],
        out_specs=pl.BlockSpec((_BE2, D), lambda i: (i, 0)),
        out_shape=jax.ShapeDtypeStruct((E // 2, D), jnp.float32),
    )(ef3, ef3, w1, b1, w2, b2, pa, pb)


@functools.partial(
    pl.kernel,
    out_type=jax.ShapeDtypeStruct((NC, NP, D), jnp.float32),
    mesh=_sc_mesh,
    scratch_types=[
        pltpu.VMEM_SHARED((NP, D), jnp.float32),
        pltpu.VMEM((C, D), jnp.float32),
        pltpu.VMEM((C, D), jnp.float32),
        pltpu.VMEM((C, D), jnp.float32),
        pltpu.VMEM((C, D), jnp.float32),
        pltpu.VMEM((C,), jnp.int32),
        pltpu.VMEM((C,), jnp.int32),
        pltpu.VMEM((C,), jnp.int32),
        pltpu.VMEM((C,), jnp.int32),
        pltpu.SemaphoreType.DMA,
        pltpu.SemaphoreType.DMA,
        pltpu.SemaphoreType.DMA,
        pltpu.SemaphoreType.DMA,
        pltpu.SemaphoreType.DMA,
        pltpu.SemaphoreType.DMA,
        pltpu.SemaphoreType.DMA,
        pltpu.SemaphoreType.DMA,
        pltpu.SemaphoreType.DMA,
        pltpu.SemaphoreType.DMA,
    ],
)
def _sc_pass(
    e_hbm, h_hbm, src_hbm, dst_hbm, out_hbm,
    aggr, ebuf0, ebuf1, hbuf0, hbuf1,
    sibuf0, sibuf1, dbuf0, dbuf1,
    seme0, seme1, semg0, semg1, sems0, sems1, semi0, semi1, semd0, semd1,
):
    c = lax.axis_index("c")
    s = lax.axis_index("s")
    wid = s * NC + c

    ebufs = (ebuf0, ebuf1)
    hbufs = (hbuf0, hbuf1)
    sibufs = (sibuf0, sibuf1)
    dbufs = (dbuf0, dbuf1)
    semes = (seme0, seme1)
    semgs = (semg0, semg1)
    semss = (sems0, sems1)
    semis = (semi0, semi1)
    semds = (semd0, semd1)

    def _zrow(i, carry):
        for j in range(D // L):
            hbuf0[i, pl.ds(j * L, L)] = jnp.zeros((L,), jnp.float32)
        return carry

    lax.fori_loop(0, C, _zrow, 0)
    zbase = s * RPT
    for k in range(RPT // C):
        pltpu.sync_copy(hbuf0, aggr.at[pl.ds(zbase + k * C, C)])
    plsc.subcore_barrier()

    pbase = wid * (EPT // 2)

    def _lo(g):
        return pl.ds(pbase + g * (C // 2), C // 2)

    def _hi(g):
        return pl.ds(pbase + g * (C // 2) + E // 2, C // 2)

    def _idx_dma(g, p):
        pltpu.async_copy(src_hbm.at[_lo(g)], sibufs[p].at[pl.ds(0, C // 2)], semis[p])
        pltpu.async_copy(
            src_hbm.at[_hi(g)], sibufs[p].at[pl.ds(C // 2, C // 2)], semis[p]
        )

    def _issue(g, p, wait_scatter):
        pltpu.make_async_copy(
            src_hbm.at[_lo(g)], sibufs[p].at[pl.ds(0, C // 2)], semis[p]
        ).wait()
        pltpu.make_async_copy(
            src_hbm.at[_hi(g)], sibufs[p].at[pl.ds(C // 2, C // 2)], semis[p]
        ).wait()
        pltpu.async_copy(h_hbm.at[sibufs[p]], hbufs[p], semgs[p])
        if wait_scatter:
            pltpu.make_async_copy(ebufs[p], aggr.at[dbufs[p]], semss[p]).wait()
        pltpu.async_copy(dst_hbm.at[_lo(g)], dbufs[p].at[pl.ds(0, C // 2)], semds[p])
        pltpu.async_copy(
            dst_hbm.at[_hi(g)], dbufs[p].at[pl.ds(C // 2, C // 2)], semds[p]
        )
        pltpu.async_copy(
            e_hbm.at[pl.ds(pbase + g * (C // 2), C // 2)],
            ebufs[p].at[pl.ds(C // 2, C // 2)],
            semes[p],
        )

    def _compute(g, p, prefetch_g):
        pltpu.make_async_copy(
            e_hbm.at[pl.ds(pbase + g * (C // 2), C // 2)],
            ebufs[p].at[pl.ds(C // 2, C // 2)],
            semes[p],
        ).wait()
        pltpu.make_async_copy(h_hbm.at[sibufs[p]], hbufs[p], semgs[p]).wait()
        if prefetch_g is not None:
            _idx_dma(prefetch_g, p)

        def _rowpair(ii, inner):
            ve = [
                lax.bitcast_convert_type(
                    ebufs[p][C // 2 + ii, pl.ds(L * t, L)], jnp.int32
                )
                for t in range(2 * (D // 32))
            ]
            res = []
            for r in range(2):
                i = ii + r * (C // 2)
                for t in range(D // 32):
                    v = ve[4 * r + t]
                    lo = hbufs[p][i, pl.ds(32 * t, L)] + lax.bitcast_convert_type(
                        v << 16, jnp.float32
                    )
                    hi = hbufs[p][i, pl.ds(32 * t + L, L)] + lax.bitcast_convert_type(
                        v & jnp.int32(-65536), jnp.float32
                    )
                    res.append((i, t, jnp.maximum(lo, 0.0), jnp.maximum(hi, 0.0)))
            for i, t, vlo, vhi in res:
                ebufs[p][i, pl.ds(32 * t, L)] = vlo
                ebufs[p][i, pl.ds(32 * t + L, L)] = vhi
            return inner

        lax.fori_loop(0, C // 2, _rowpair, 0)
        pltpu.make_async_copy(
            dst_hbm.at[_lo(g)], dbufs[p].at[pl.ds(0, C // 2)], semds[p]
        ).wait()
        pltpu.make_async_copy(
            dst_hbm.at[_hi(g)], dbufs[p].at[pl.ds(C // 2, C // 2)], semds[p]
        ).wait()
        pltpu.async_copy(ebufs[p], aggr.at[dbufs[p]], semss[p], add=True)

    _idx_dma(0, 0)
    _idx_dma(1, 1)
    _issue(0, 0, False)
    _issue(1, 1, False)
    _compute(0, 0, 2)
    _issue(2, 0, True)
    _compute(1, 1, 3)

    def _pair(k, carry):
        g = 2 * k
        _issue(g + 1, 1, True)
        _compute(g, 0, g + 2)
        _issue(g + 2, 0, True)
        _compute(g + 1, 1, g + 3)
        return carry

    lax.fori_loop(1, (NCHUNK - 1) // 2 - 1, _pair, 0)
    _issue(NCHUNK - 2, 1, True)
    _compute(NCHUNK - 3, 0, NCHUNK - 1)
    _issue(NCHUNK - 1, 0, True)
    _compute(NCHUNK - 2, 1, None)
    _compute(NCHUNK - 1, 0, None)
    pltpu.make_async_copy(ebuf0, aggr.at[dbuf0], sems0).wait()
    pltpu.make_async_copy(ebuf1, aggr.at[dbuf1], sems1).wait()
    plsc.subcore_barrier()

    pltpu.sync_copy(
        aggr.at[pl.ds(zbase, RPT)], out_hbm.at[c, pl.ds(zbase, RPT), :]
    )


_BN = 1000


def _apply_body(h_ref, p_ref, w_ref, b_ref, o_ref):
    x = (1.0 + EPS) * h_ref[...] + p_ref[0] + p_ref[1]
    o_ref[...] = jnp.maximum(
        jnp.dot(x, w_ref[...], preferred_element_type=jnp.float32) + b_ref[...], 0.0
    )


def _apply(h, partials, w, b):
    return pl.pallas_call(
        _apply_body,
        grid=(N // _BN,),
        in_specs=[
            pl.BlockSpec((_BN, D), lambda i: (i, 0)),
            pl.BlockSpec((NC, _BN, D), lambda i: (0, i, 0)),
            pl.BlockSpec((D, D), lambda i: (0, 0)),
            pl.BlockSpec((1, D), lambda i: (0, 0)),
        ],
        out_specs=pl.BlockSpec((_BN, D), lambda i: (i, 0)),
        out_shape=jax.ShapeDtypeStruct((N, D), jnp.float32),
    )(h, partials, w, b)


def kernel(node_feats, edge_feats, edge_index, We1, be1, We2, be2, W0, b0, W1, b1):
    src = edge_index[0]
    dst = edge_index[1]
    eye = np.eye(D, dtype=np.float32)
    e32 = _edge_mlp(
        edge_feats.T,
        We1.astype(jnp.bfloat16),
        be1.reshape(1, D),
        We2.astype(jnp.bfloat16),
        be2.reshape(1, D),
        jnp.asarray(eye[:, : D // 2], dtype=jnp.bfloat16),
        jnp.asarray(eye[:, D // 2 :], dtype=jnp.bfloat16),
    )
    q = jnp.asarray(_Q)
    h = node_feats[:, q]
    layer_params = (
        (W0[q][:, q], b0[q]),
        (W1[q], b1),
    )
    for w, b in layer_params:
        partials = _sc_pass(e32, h, src, dst)
        h = _apply(h, partials, w, b.reshape(1, D))
    return h


# speedup vs baseline: 1.5201x; 1.0413x over previous
import functools

import numpy as np
import jax
import jax.numpy as jnp
from jax import lax
from jax.experimental import pallas as pl
from jax.experimental.pallas import tpu as pltpu
from jax.experimental.pallas import tpu_sc as plsc

N = 10000
E = 320000
D = 128
DE = 16
EPS = 0.0

NC = 2
NS = 16
L = 16
NW = NC * NS
EPT = E // NW
C = 80
NCHUNK = EPT // C
NP = 10240
RPT = NP // NS

_Q = np.zeros(D, dtype=np.int32)
for _t in range(D // 32):
    for _k in range(16):
        _Q[32 * _t + _k] = 16 * _t + _k
        _Q[32 * _t + 16 + _k] = D // 2 + 16 * _t + _k

_sc_mesh = plsc.VectorSubcoreMesh(
    core_axis_name="c", subcore_axis_name="s", num_cores=NC, num_subcores=NS
)


_BE2 = 6400


def _mlp_one(xt, w1, b1, w2, b2, pa, pb):
    t = jnp.maximum(
        lax.dot_general(
            xt.astype(jnp.bfloat16),
            w1,
            (((0,), (0,)), ((), ())),
            preferred_element_type=jnp.float32,
        )
        + b1,
        0.0,
    )
    y = (
        jnp.dot(t.astype(jnp.bfloat16), w2, preferred_element_type=jnp.float32) + b2
    ).astype(jnp.bfloat16)
    ya = jnp.dot(y, pa, preferred_element_type=jnp.float32).astype(jnp.bfloat16)
    yb = jnp.dot(y, pb, preferred_element_type=jnp.float32).astype(jnp.bfloat16)
    ua = jax.lax.bitcast_convert_type(ya, jnp.uint16).astype(jnp.int32)
    ub = jax.lax.bitcast_convert_type(yb, jnp.uint16).astype(jnp.int32)
    return ua | (ub << 16)


def _edge_mlp_body(xe_ref, xo_ref, w1_ref, b1_ref, w2_ref, b2_ref, pa_ref, pb_ref, o_ref):
    w1 = w1_ref[...]
    b1 = b1_ref[...]
    w2 = w2_ref[...]
    b2 = b2_ref[...]
    pa = pa_ref[...]
    pb = pb_ref[...]
    pe = _mlp_one(xe_ref[...], w1, b1, w2, b2, pa, pb)
    po = _mlp_one(xo_ref[...], w1, b1, w2, b2, pa, pb)
    o_ref[...] = jax.lax.bitcast_convert_type(
        jnp.concatenate([pe, po], axis=1), jnp.float32
    )


def _edge_mlp(ef3, w1, b1, w2, b2, pa, pb):
    return pl.pallas_call(
        _edge_mlp_body,
        grid=(E // 2 // _BE2,),
        in_specs=[
            pl.BlockSpec((DE, _BE2), lambda i: (0, i)),
            pl.BlockSpec((DE, _BE2), lambda i: (0, i + E // 2 // _BE2)),
            pl.BlockSpec((DE, D), lambda i: (0, 0)),
            pl.BlockSpec((1, D), lambda i: (0, 0)),
            pl.BlockSpec((D, D), lambda i: (0, 0)),
            pl.BlockSpec((1, D), lambda i: (0, 0)),
            pl.BlockSpec((D, D // 2), lambda i: (0, 0)),
            pl.BlockSpec((D, D // 2), lambda i: (0, 0)),
        ],
        out_specs=pl.BlockSpec((_BE2, D), lambda i: (i, 0)),
        out_shape=jax.ShapeDtypeStruct((E // 2, D), jnp.float32),
    )(ef3, ef3, w1, b1, w2, b2, pa, pb)


@functools.partial(
    pl.kernel,
    out_type=jax.ShapeDtypeStruct((NC, NP, D), jnp.float32),
    mesh=_sc_mesh,
    scratch_types=[
        pltpu.VMEM_SHARED((NP, D), jnp.float32),
        pltpu.VMEM((C, D), jnp.float32),
        pltpu.VMEM((C, D), jnp.float32),
        pltpu.VMEM((C, D), jnp.float32),
        pltpu.VMEM((C, D), jnp.float32),
        pltpu.VMEM((C,), jnp.int32),
        pltpu.VMEM((C,), jnp.int32),
        pltpu.VMEM((C,), jnp.int32),
        pltpu.VMEM((C,), jnp.int32),
        pltpu.SemaphoreType.DMA,
        pltpu.SemaphoreType.DMA,
        pltpu.SemaphoreType.DMA,
        pltpu.SemaphoreType.DMA,
        pltpu.SemaphoreType.DMA,
        pltpu.SemaphoreType.DMA,
        pltpu.SemaphoreType.DMA,
        pltpu.SemaphoreType.DMA,
        pltpu.SemaphoreType.DMA,
        pltpu.SemaphoreType.DMA,
    ],
)
def _sc_pass(
    e_hbm, h_hbm, src_hbm, dst_hbm, out_hbm,
    aggr, ebuf0, ebuf1, hbuf0, hbuf1,
    sibuf0, sibuf1, dbuf0, dbuf1,
    seme0, seme1, semg0, semg1, sems0, sems1, semi0, semi1, semd0, semd1,
):
    c = lax.axis_index("c")
    s = lax.axis_index("s")
    wid = s * NC + c

    ebufs = (ebuf0, ebuf1)
    hbufs = (hbuf0, hbuf1)
    sibufs = (sibuf0, sibuf1)
    dbufs = (dbuf0, dbuf1)
    semes = (seme0, seme1)
    semgs = (semg0, semg1)
    semss = (sems0, sems1)
    semis = (semi0, semi1)
    semds = (semd0, semd1)

    def _zrow(i, carry):
        for j in range(D // L):
            hbuf0[i, pl.ds(j * L, L)] = jnp.zeros((L,), jnp.float32)
        return carry

    lax.fori_loop(0, C, _zrow, 0)
    zbase = s * RPT
    for k in range(RPT // C):
        pltpu.sync_copy(hbuf0, aggr.at[pl.ds(zbase + k * C, C)])
    plsc.subcore_barrier()

    pbase = wid * (EPT // 2)

    def _lo(g):
        return pl.ds(pbase + g * (C // 2), C // 2)

    def _hi(g):
        return pl.ds(pbase + g * (C // 2) + E // 2, C // 2)

    def _dlo(g):
        return pl.ds(E + pbase + g * (C // 2), C // 2)

    def _dhi(g):
        return pl.ds(E + pbase + g * (C // 2) + E // 2, C // 2)

    def _idx_dma(g, p):
        pltpu.async_copy(src_hbm.at[_lo(g)], sibufs[p].at[pl.ds(0, C // 2)], semis[p])
        pltpu.async_copy(
            src_hbm.at[_hi(g)], sibufs[p].at[pl.ds(C // 2, C // 2)], semis[p]
        )

    def _issue(g, p, wait_scatter):
        pltpu.make_async_copy(
            src_hbm.at[_lo(g)], sibufs[p].at[pl.ds(0, C // 2)], semis[p]
        ).wait()
        pltpu.make_async_copy(
            src_hbm.at[_hi(g)], sibufs[p].at[pl.ds(C // 2, C // 2)], semis[p]
        ).wait()
        pltpu.async_copy(h_hbm.at[sibufs[p]], hbufs[p], semgs[p])
        if wait_scatter:
            pltpu.make_async_copy(ebufs[p], aggr.at[dbufs[p]], semss[p]).wait()
        pltpu.async_copy(dst_hbm.at[_dlo(g)], dbufs[p].at[pl.ds(0, C // 2)], semds[p])
        pltpu.async_copy(
            dst_hbm.at[_dhi(g)], dbufs[p].at[pl.ds(C // 2, C // 2)], semds[p]
        )
        pltpu.async_copy(
            e_hbm.at[pl.ds(pbase + g * (C // 2), C // 2)],
            ebufs[p].at[pl.ds(C // 2, C // 2)],
            semes[p],
        )

    def _compute(g, p, prefetch_g):
        pltpu.make_async_copy(
            e_hbm.at[pl.ds(pbase + g * (C // 2), C // 2)],
            ebufs[p].at[pl.ds(C // 2, C // 2)],
            semes[p],
        ).wait()
        pltpu.make_async_copy(h_hbm.at[sibufs[p]], hbufs[p], semgs[p]).wait()
        if prefetch_g is not None:
            _idx_dma(prefetch_g, p)

        def _rowpair(ii, inner):
            ve = [
                lax.bitcast_convert_type(
                    ebufs[p][C // 2 + ii, pl.ds(L * t, L)], jnp.int32
                )
                for t in range(2 * (D // 32))
            ]
            res = []
            for r in range(2):
                i = ii + r * (C // 2)
                for t in range(D // 32):
                    v = ve[4 * r + t]
                    lo = hbufs[p][i, pl.ds(L * t, L)] + lax.bitcast_convert_type(
                        v << 16, jnp.float32
                    )
                    hi = hbufs[p][i, pl.ds(D // 2 + L * t, L)] + lax.bitcast_convert_type(
                        v & jnp.int32(-65536), jnp.float32
                    )
                    res.append((i, t, jnp.maximum(lo, 0.0), jnp.maximum(hi, 0.0)))
            for i, t, vlo, vhi in res:
                ebufs[p][i, pl.ds(32 * t, L)] = vlo
                ebufs[p][i, pl.ds(32 * t + L, L)] = vhi
            return inner

        lax.fori_loop(0, C // 2, _rowpair, 0)
        pltpu.make_async_copy(
            dst_hbm.at[_dlo(g)], dbufs[p].at[pl.ds(0, C // 2)], semds[p]
        ).wait()
        pltpu.make_async_copy(
            dst_hbm.at[_dhi(g)], dbufs[p].at[pl.ds(C // 2, C // 2)], semds[p]
        ).wait()
        pltpu.async_copy(ebufs[p], aggr.at[dbufs[p]], semss[p], add=True)

    _idx_dma(0, 0)
    _idx_dma(1, 1)
    _issue(0, 0, False)
    _issue(1, 1, False)
    _compute(0, 0, 2)
    _issue(2, 0, True)
    _compute(1, 1, 3)

    def _pair(k, carry):
        g = 2 * k
        _issue(g + 1, 1, True)
        _compute(g, 0, g + 2)
        _issue(g + 2, 0, True)
        _compute(g + 1, 1, g + 3)
        return carry

    lax.fori_loop(1, (NCHUNK - 1) // 2 - 1, _pair, 0)
    _issue(NCHUNK - 2, 1, True)
    _compute(NCHUNK - 3, 0, NCHUNK - 1)
    _issue(NCHUNK - 1, 0, True)
    _compute(NCHUNK - 2, 1, None)
    _compute(NCHUNK - 1, 0, None)
    pltpu.make_async_copy(ebuf0, aggr.at[dbuf0], sems0).wait()
    pltpu.make_async_copy(ebuf1, aggr.at[dbuf1], sems1).wait()
    plsc.subcore_barrier()

    pltpu.sync_copy(
        aggr.at[pl.ds(zbase, RPT)], out_hbm.at[c, pl.ds(zbase, RPT), :]
    )


_BN = 1000


def _apply_body(h_ref, p_ref, w_ref, wq_ref, b_ref, o_ref):
    y = (
        jnp.dot(
            (1.0 + EPS) * h_ref[...], w_ref[...], preferred_element_type=jnp.float32
        )
        + jnp.dot(
            p_ref[0] + p_ref[1], wq_ref[...], preferred_element_type=jnp.float32
        )
        + b_ref[...]
    )
    o_ref[...] = jnp.maximum(y, 0.0)


def _apply(h, partials, w, wq, b):
    return pl.pallas_call(
        _apply_body,
        grid=(N // _BN,),
        in_specs=[
            pl.BlockSpec((_BN, D), lambda i: (i, 0)),
            pl.BlockSpec((NC, _BN, D), lambda i: (0, i, 0)),
            pl.BlockSpec((D, D), lambda i: (0, 0)),
            pl.BlockSpec((D, D), lambda i: (0, 0)),
            pl.BlockSpec((1, D), lambda i: (0, 0)),
        ],
        out_specs=pl.BlockSpec((_BN, D), lambda i: (i, 0)),
        out_shape=jax.ShapeDtypeStruct((N, D), jnp.float32),
    )(h, partials, w, wq, b)


def kernel(node_feats, edge_feats, edge_index, We1, be1, We2, be2, W0, b0, W1, b1):
    ei = edge_index.reshape(2 * E)
    eye = np.eye(D, dtype=np.float32)
    e32 = _edge_mlp(
        edge_feats.T,
        We1.astype(jnp.bfloat16),
        be1.reshape(1, D),
        We2.astype(jnp.bfloat16),
        be2.reshape(1, D),
        jnp.asarray(eye[:, : D // 2], dtype=jnp.bfloat16),
        jnp.asarray(eye[:, D // 2 :], dtype=jnp.bfloat16),
    )
    q = jnp.asarray(_Q)
    h = node_feats
    for w, b in ((W0, b0), (W1, b1)):
        partials = _sc_pass(e32, h, ei, ei)
        h = _apply(h, partials, w, w[q], b.reshape(1, D))
    return h
